# baseline (speedup 1.0000x reference)
import functools

import jax
import jax.numpy as jnp
from jax import lax
from jax.experimental import pallas as pl
from jax.experimental.pallas import tpu as pltpu
from jax.experimental.pallas import tpu_sc as plsc

N = 131072
C = 20
K = 2048
NTILES = 32
PT = N // NTILES
SUB = 512
NSUB = PT // SUB


def _sc_body(preds_hbm, labels_hbm, out_hbm, pbuf, lbuf, hist):
    wid = lax.axis_index("c") * 16 + lax.axis_index("s")

    zeros16 = jnp.zeros((16,), jnp.float32)
    ones16 = jnp.ones((16,), jnp.float32)

    def _zero(i, carry):
        hist[pl.ds(i * 16, 16)] = zeros16
        return carry
    lax.fori_loop(0, 2 * C * K // 16, _zero, 0)

    lane = lax.iota(jnp.int32, 16)

    def _group(g, carry):
        p0 = g * 16
        lab = lbuf[pl.ds(p0, 16)]
        rows = p0 + lane
        v = [plsc.load_gather(pbuf, [rows, jnp.full((16,), c, jnp.int32)])
             for c in range(C)]
        m = v[0]
        for c in range(1, C):
            m = jnp.maximum(m, v[c])
        t = [jnp.exp(v[c] - m) for c in range(C)]
        s = t[0]
        for c in range(1, C):
            s = s + t[c]
        rinv = 1.0 / s
        fgbin = jnp.zeros((16,), jnp.int32)
        for c in range(C):
            p = t[c] * rinv
            err = jnp.abs(1.0 - p)
            b = (err * K).astype(jnp.int32)
            b = jnp.minimum(jnp.maximum(b, 0), K - 1)
            plsc.addupdate_scatter(hist, [b + c * K], ones16)
            fgbin = jnp.where(lab == c, b, fgbin)
        plsc.addupdate_scatter(hist, [(lab + C) * K + fgbin], ones16)
        return carry

    for sidx in range(NSUB):
        base = wid * PT + sidx * SUB
        pltpu.sync_copy(preds_hbm.at[pl.ds(base, SUB)], pbuf)
        pltpu.sync_copy(labels_hbm.at[pl.ds(base, SUB)], lbuf)
        lax.fori_loop(0, SUB // 16, _group, 0)

    for r in range(2 * C):
        pltpu.sync_copy(hist.at[pl.ds(r * K, K)], out_hbm.at[wid, r])


@functools.partial(
    pl.kernel,
    out_type=jax.ShapeDtypeStruct((NTILES, 2 * C, K), jnp.float32),
    mesh=plsc.VectorSubcoreMesh(core_axis_name="c", subcore_axis_name="s"),
    compiler_params=pltpu.CompilerParams(
        needs_layout_passes=False, use_tc_tiling_on_sc=False),
    scratch_types=[
        pltpu.VMEM((SUB, C), jnp.float32),
        pltpu.VMEM((SUB,), jnp.int32),
        pltpu.VMEM((2 * C * K,), jnp.float32),
    ],
)
def _sc_hist(preds_hbm, labels_hbm, out_hbm, pbuf, lbuf, hist):
    _sc_body(preds_hbm, labels_hbm, out_hbm, pbuf, lbuf, hist)


def _tc_body(hist_ref, out_ref):
    tot = jnp.sum(hist_ref[...], axis=0)
    cnt = tot[:C, :]
    fg = tot[C:, :]
    ir = lax.broadcasted_iota(jnp.int32, (K, K), 0)
    ic = lax.broadcasted_iota(jnp.int32, (K, K), 1)
    M = (ir >= ic).astype(jnp.float32)
    dn = (((1,), (0,)), ((), ()))
    Nk = lax.dot_general(cnt, M, dn, preferred_element_type=jnp.float32)
    Fk = lax.dot_general(fg, M, dn, preferred_element_type=jnp.float32)
    G = Fk[:, 0:1]
    denom = G + Nk - Fk
    I = jnp.where(denom > 0, 1.0 - (G - Fk) / denom, 0.0)
    loss_c = (jnp.sum(I, axis=1, keepdims=True) - 0.5 * I[:, 0:1]) * (1.0 / K)
    present = (G > 0).astype(jnp.float32)
    loss = jnp.sum(loss_c * present) / jnp.maximum(jnp.sum(present), 1.0)
    out_ref[...] = jnp.broadcast_to(loss, (1, 1))


def _tc_finish(hist):
    return pl.pallas_call(
        _tc_body,
        out_shape=jax.ShapeDtypeStruct((1, 1), jnp.float32),
    )(hist)


def kernel(preds, labels):
    labels = labels.astype(jnp.int32)
    hist = _sc_hist(preds, labels)
    return _tc_finish(hist)[0, 0]


# speedup vs baseline: 1.3360x; 1.3360x over previous
import functools

import jax
import jax.numpy as jnp
from jax import lax
from jax.experimental import pallas as pl
from jax.experimental.pallas import tpu as pltpu
from jax.experimental.pallas import tpu_sc as plsc

N = 131072
C = 20
K = 1024
R = 48
NTILES = 32
PT = N // NTILES
SUB = 512
NSUB = PT // SUB


def _sc_body(preds_hbm, labels_hbm, out_hbm, pbuf, lbuf, hist, rowidx, shared):
    core = lax.axis_index("c")
    sid = lax.axis_index("s")

    zeros16 = jnp.zeros((16,), jnp.float32)
    ones16 = jnp.ones((16,), jnp.float32)
    lane = lax.iota(jnp.int32, 16)

    def _zrow(i, carry):
        def _zcol(j, carry2):
            hist[i, pl.ds(j * 16, 16)] = zeros16
            return carry2
        return lax.fori_loop(0, K // 16, _zcol, carry)
    lax.fori_loop(0, R, _zrow, 0)
    for j in range(R // 16):
        rowidx[pl.ds(j * 16, 16)] = lane + (j * 16)

    @pl.when(sid == 0)
    def _():
        pltpu.sync_copy(hist, shared)
    plsc.subcore_barrier()

    def _group(g, carry):
        p0 = g * 16
        lab = lbuf[pl.ds(p0, 16)]
        rbase = (p0 + lane) * C
        v = [plsc.load_gather(pbuf, [rbase + c]) for c in range(C)]
        m = v[0]
        for c in range(1, C):
            m = jnp.maximum(m, v[c])
        t = [jnp.exp(v[c] - m) for c in range(C)]
        s = t[0]
        for c in range(1, C):
            s = s + t[c]
        rinv = 1.0 / s
        fgbin = jnp.zeros((16,), jnp.int32)
        for c in range(C):
            p = t[c] * rinv
            err = jnp.abs(1.0 - p)
            b = (err * K).astype(jnp.int32)
            b = jnp.minimum(jnp.maximum(b, 0), K - 1)
            plsc.addupdate_scatter(
                hist, [jnp.full((16,), c, jnp.int32), b], ones16)
            fgbin = jnp.where(lab == c, b, fgbin)
        plsc.addupdate_scatter(hist, [lab + C, fgbin], ones16)
        return carry

    wid = core * 16 + sid
    for sidx in range(NSUB):
        base = wid * PT + sidx * SUB
        pltpu.sync_copy(preds_hbm.at[pl.ds(base * C, SUB * C)], pbuf)
        pltpu.sync_copy(labels_hbm.at[pl.ds(base, SUB)], lbuf)
        lax.fori_loop(0, SUB // 16, _group, 0)

    pltpu.sync_copy(hist, shared.at[rowidx], add=True)
    plsc.subcore_barrier()
    @pl.when(sid == 0)
    def _():
        pltpu.sync_copy(shared, out_hbm.at[core])


@functools.partial(
    pl.kernel,
    out_type=jax.ShapeDtypeStruct((2, R, K), jnp.float32),
    mesh=plsc.VectorSubcoreMesh(core_axis_name="c", subcore_axis_name="s"),
    compiler_params=pltpu.CompilerParams(
        needs_layout_passes=False, use_tc_tiling_on_sc=False),
    scratch_types=[
        pltpu.VMEM((SUB * C,), jnp.float32),
        pltpu.VMEM((SUB,), jnp.int32),
        pltpu.VMEM((R, K), jnp.float32),
        pltpu.VMEM((R,), jnp.int32),
        pltpu.VMEM_SHARED((R, K), jnp.float32),
    ],
)
def _sc_hist(preds_hbm, labels_hbm, out_hbm, pbuf, lbuf, hist, rowidx, shared):
    _sc_body(preds_hbm, labels_hbm, out_hbm, pbuf, lbuf, hist, rowidx, shared)


def _tc_body(hist_ref, out_ref):
    tot = jnp.sum(hist_ref[...], axis=0)
    cnt = tot[:C, :]
    fg = tot[C:2 * C, :]
    ir = lax.broadcasted_iota(jnp.int32, (K, K), 0)
    ic = lax.broadcasted_iota(jnp.int32, (K, K), 1)
    M = (ir >= ic).astype(jnp.float32)
    dn = (((1,), (0,)), ((), ()))
    Nk = lax.dot_general(cnt, M, dn, preferred_element_type=jnp.float32)
    Fk = lax.dot_general(fg, M, dn, preferred_element_type=jnp.float32)
    G = Fk[:, 0:1]
    denom = G + Nk - Fk
    I = jnp.where(denom > 0, 1.0 - (G - Fk) / denom, 0.0)
    loss_c = (jnp.sum(I, axis=1, keepdims=True) - 0.5 * I[:, 0:1]) * (1.0 / K)
    present = (G > 0).astype(jnp.float32)
    loss = jnp.sum(loss_c * present) / jnp.maximum(jnp.sum(present), 1.0)
    out_ref[...] = jnp.broadcast_to(loss, (1, 1))


def _tc_finish(hist):
    return pl.pallas_call(
        _tc_body,
        out_shape=jax.ShapeDtypeStruct((1, 1), jnp.float32),
    )(hist)


def kernel(preds, labels):
    labels = labels.astype(jnp.int32)
    hist = _sc_hist(preds.reshape(-1), labels)
    return _tc_finish(hist)[0, 0]


# speedup vs baseline: 2.5351x; 1.8975x over previous
import functools

import jax
import jax.numpy as jnp
from jax import lax
from jax.experimental import pallas as pl
from jax.experimental.pallas import tpu as pltpu
from jax.experimental.pallas import tpu_sc as plsc

N = 131072
C = 20
K = 1024
R = 48
NTILES = 32
PT = N // NTILES
SUB = 1024
NSUB = PT // SUB


def _sc_body(predsT_hbm, labels_hbm, out_hbm, pbuf, lbuf, hist, rowidx, shared):
    core = lax.axis_index("c")
    sid = lax.axis_index("s")

    zeros16 = jnp.zeros((16,), jnp.float32)
    ones16 = jnp.ones((16,), jnp.float32)
    lane = lax.iota(jnp.int32, 16)

    def _zrow(i, carry):
        def _zcol(j, carry2):
            hist[i, pl.ds(j * 16, 16)] = zeros16
            return carry2
        return lax.fori_loop(0, K // 16, _zcol, carry)
    lax.fori_loop(0, R, _zrow, 0)
    for j in range(R // 16):
        rowidx[pl.ds(j * 16, 16)] = lane + (j * 16)

    @pl.when(sid == 0)
    def _():
        pltpu.sync_copy(hist, shared)
    plsc.subcore_barrier()

    def _group(g, carry):
        p0 = g * 16
        lab = lbuf[pl.ds(p0, 16)]
        v = [pbuf[c, pl.ds(p0, 16)] for c in range(C)]
        m = v[0]
        for c in range(1, C):
            m = jnp.maximum(m, v[c])
        t = [jnp.exp(v[c] - m) for c in range(C)]
        s = t[0]
        for c in range(1, C):
            s = s + t[c]
        rinv = 1.0 / s
        fgbin = jnp.zeros((16,), jnp.int32)
        for c in range(C):
            p = t[c] * rinv
            err = jnp.abs(1.0 - p)
            b = (err * K).astype(jnp.int32)
            b = jnp.minimum(jnp.maximum(b, 0), K - 1)
            plsc.addupdate_scatter(
                hist, [jnp.full((16,), c, jnp.int32), b], ones16)
            fgbin = jnp.where(lab == c, b, fgbin)
        plsc.addupdate_scatter(hist, [lab + C, fgbin], ones16)
        return carry

    wid = core * 16 + sid
    for sidx in range(NSUB):
        base = wid * PT + sidx * SUB
        pltpu.sync_copy(predsT_hbm.at[:, pl.ds(base, SUB)], pbuf)
        pltpu.sync_copy(labels_hbm.at[pl.ds(base, SUB)], lbuf)
        lax.fori_loop(0, SUB // 16, _group, 0)

    pltpu.sync_copy(hist, shared.at[rowidx], add=True)
    plsc.subcore_barrier()
    @pl.when(sid == 0)
    def _():
        pltpu.sync_copy(shared, out_hbm.at[core])


@functools.partial(
    pl.kernel,
    out_type=jax.ShapeDtypeStruct((2, R, K), jnp.float32),
    mesh=plsc.VectorSubcoreMesh(core_axis_name="c", subcore_axis_name="s"),
    compiler_params=pltpu.CompilerParams(
        needs_layout_passes=False, use_tc_tiling_on_sc=False),
    scratch_types=[
        pltpu.VMEM((C, SUB), jnp.float32),
        pltpu.VMEM((SUB,), jnp.int32),
        pltpu.VMEM((R, K), jnp.float32),
        pltpu.VMEM((R,), jnp.int32),
        pltpu.VMEM_SHARED((R, K), jnp.float32),
    ],
)
def _sc_hist(predsT_hbm, labels_hbm, out_hbm, pbuf, lbuf, hist, rowidx, shared):
    _sc_body(predsT_hbm, labels_hbm, out_hbm, pbuf, lbuf, hist, rowidx, shared)


def _tc_body(hist_ref, out_ref):
    tot = jnp.sum(hist_ref[...], axis=0)
    cnt = tot[:C, :]
    fg = tot[C:2 * C, :]
    ir = lax.broadcasted_iota(jnp.int32, (K, K), 0)
    ic = lax.broadcasted_iota(jnp.int32, (K, K), 1)
    M = (ir >= ic).astype(jnp.float32)
    dn = (((1,), (0,)), ((), ()))
    Nk = lax.dot_general(cnt, M, dn, preferred_element_type=jnp.float32)
    Fk = lax.dot_general(fg, M, dn, preferred_element_type=jnp.float32)
    G = Fk[:, 0:1]
    denom = G + Nk - Fk
    I = jnp.where(denom > 0, 1.0 - (G - Fk) / denom, 0.0)
    loss_c = (jnp.sum(I, axis=1, keepdims=True) - 0.5 * I[:, 0:1]) * (1.0 / K)
    present = (G > 0).astype(jnp.float32)
    loss = jnp.sum(loss_c * present) / jnp.maximum(jnp.sum(present), 1.0)
    out_ref[...] = jnp.broadcast_to(loss, (1, 1))


def _tc_finish(hist):
    return pl.pallas_call(
        _tc_body,
        out_shape=jax.ShapeDtypeStruct((1, 1), jnp.float32),
    )(hist)


def kernel(preds, labels):
    labels = labels.astype(jnp.int32)
    hist = _sc_hist(jnp.swapaxes(preds, 0, 1), labels)
    return _tc_finish(hist)[0, 0]


# speedup vs baseline: 3.3346x; 1.3154x over previous
import functools

import jax
import jax.numpy as jnp
from jax import lax
from jax.experimental import pallas as pl
from jax.experimental.pallas import tpu as pltpu
from jax.experimental.pallas import tpu_sc as plsc

N = 131072
C = 20
K = 1024
R = 48
NTILES = 32
PT = N // NTILES
SUB = 1024
NSUB = PT // SUB


def _tree(xs, op):
    xs = list(xs)
    while len(xs) > 1:
        nxt = [op(xs[i], xs[i + 1]) for i in range(0, len(xs) - 1, 2)]
        if len(xs) % 2:
            nxt.append(xs[-1])
        xs = nxt
    return xs[0]


def _sc_body(predsT_hbm, labels_hbm, out_hbm, pbuf, lbuf, hist, rowidx,
             shared, semp, seml):
    core = lax.axis_index("c")
    sid = lax.axis_index("s")

    zeros16 = jnp.zeros((16,), jnp.float32)
    ones16 = jnp.ones((16,), jnp.float32)
    lane = lax.iota(jnp.int32, 16)

    def _zrow(i, carry):
        for j in range(K // 16):
            hist[i, pl.ds(j * 16, 16)] = zeros16
        return carry
    lax.fori_loop(0, R, _zrow, 0)
    for j in range(R // 16):
        rowidx[pl.ds(j * 16, 16)] = lane + (j * 16)

    @pl.when(sid == 0)
    def _():
        pltpu.sync_copy(hist, shared)

    def _one_group(p0, buf):
        lab = lbuf[buf, pl.ds(p0, 16)]
        v = [pbuf[buf, c, pl.ds(p0, 16)] for c in range(C)]
        m = _tree(v, jnp.maximum)
        t = [jnp.exp(v[c] - m) for c in range(C)]
        s = _tree(t, lambda a, b: a + b)
        rinv = 1.0 / s
        fgbin = jnp.zeros((16,), jnp.int32)
        for c in range(C):
            p = t[c] * rinv
            err = jnp.abs(1.0 - p)
            b = jnp.minimum((err * K).astype(jnp.int32), K - 1)
            plsc.addupdate_scatter(
                hist, [jnp.full((16,), c, jnp.int32), b], ones16)
            fgbin = jnp.where(lab == c, b, fgbin)
        plsc.addupdate_scatter(hist, [lab + C, fgbin], ones16)

    wid = core * 16 + sid

    def _start(s):
        b = s % 2
        base = wid * PT + s * SUB
        h1 = pltpu.async_copy(
            predsT_hbm.at[:, pl.ds(base, SUB)], pbuf.at[b], semp)
        h2 = pltpu.async_copy(
            labels_hbm.at[pl.ds(base, SUB)], lbuf.at[b], seml)
        return h1, h2

    def _group2(g, carry):
        buf = carry
        _one_group(g * 32, buf)
        _one_group(g * 32 + 16, buf)
        return carry

    h = _start(0)
    for sidx in range(NSUB):
        hn = _start(sidx + 1) if sidx + 1 < NSUB else None
        h[0].wait()
        h[1].wait()
        lax.fori_loop(0, SUB // 32, _group2, sidx % 2)
        h = hn

    plsc.subcore_barrier()
    pltpu.sync_copy(hist, shared.at[rowidx], add=True)
    plsc.subcore_barrier()
    @pl.when(sid == 0)
    def _():
        pltpu.sync_copy(shared, out_hbm.at[core])


@functools.partial(
    pl.kernel,
    out_type=jax.ShapeDtypeStruct((2, R, K), jnp.float32),
    mesh=plsc.VectorSubcoreMesh(core_axis_name="c", subcore_axis_name="s"),
    compiler_params=pltpu.CompilerParams(
        needs_layout_passes=False, use_tc_tiling_on_sc=False),
    scratch_types=[
        pltpu.VMEM((2, C, SUB), jnp.float32),
        pltpu.VMEM((2, SUB), jnp.int32),
        pltpu.VMEM((R, K), jnp.float32),
        pltpu.VMEM((R,), jnp.int32),
        pltpu.VMEM_SHARED((R, K), jnp.float32),
        pltpu.SemaphoreType.DMA,
        pltpu.SemaphoreType.DMA,
    ],
)
def _sc_hist(predsT_hbm, labels_hbm, out_hbm, pbuf, lbuf, hist, rowidx,
             shared, semp, seml):
    _sc_body(predsT_hbm, labels_hbm, out_hbm, pbuf, lbuf, hist, rowidx,
             shared, semp, seml)


def _tc_body(hist_ref, out_ref):
    tot = jnp.sum(hist_ref[...], axis=0)
    cnt = tot[:C, :]
    fg = tot[C:2 * C, :]
    ir = lax.broadcasted_iota(jnp.int32, (K, K), 0)
    ic = lax.broadcasted_iota(jnp.int32, (K, K), 1)
    M = (ir >= ic).astype(jnp.float32)
    dn = (((1,), (0,)), ((), ()))
    Nk = lax.dot_general(cnt, M, dn, preferred_element_type=jnp.float32)
    Fk = lax.dot_general(fg, M, dn, preferred_element_type=jnp.float32)
    G = Fk[:, 0:1]
    denom = G + Nk - Fk
    I = jnp.where(denom > 0, 1.0 - (G - Fk) / denom, 0.0)
    loss_c = (jnp.sum(I, axis=1, keepdims=True) - 0.5 * I[:, 0:1]) * (1.0 / K)
    present = (G > 0).astype(jnp.float32)
    loss = jnp.sum(loss_c * present) / jnp.maximum(jnp.sum(present), 1.0)
    out_ref[...] = jnp.broadcast_to(loss, (1, 1))


def _tc_finish(hist):
    return pl.pallas_call(
        _tc_body,
        out_shape=jax.ShapeDtypeStruct((1, 1), jnp.float32),
    )(hist)


def kernel(preds, labels):
    labels = labels.astype(jnp.int32)
    hist = _sc_hist(jnp.swapaxes(preds, 0, 1), labels)
    return _tc_finish(hist)[0, 0]
